# baseline (speedup 1.0000x reference)
import jax
import jax.numpy as jnp
from jax import lax
from jax.experimental import pallas as pl
from jax.experimental.pallas import tpu as pltpu
from jax.experimental.pallas import tpu_sc as plsc

B = 4096
H = 50
D = 128
C = 10
V = 100000
CP = 16
LANES = 16
DCH = D // LANES

NC = 2
NS = 16
NW = NC * NS

G = 2
RPS = G * H
SPW = B // G // NW
BPW = B // NW
NBUF = 4
NGRP = SPW // NBUF

VBLK = 10000


def _proj_body(t_ref, w_ref, o_ref):
    o_ref[...] = jnp.dot(t_ref[...], w_ref[...],
                         preferred_element_type=jnp.float32)


_proj = pl.pallas_call(
    _proj_body,
    grid=(V // VBLK,),
    in_specs=[
        pl.BlockSpec((VBLK, D), lambda i: (i, 0)),
        pl.BlockSpec((D, CP), lambda i: (0, 0)),
    ],
    out_specs=pl.BlockSpec((VBLK, CP), lambda i: (i, 0)),
    out_shape=jax.ShapeDtypeStruct((V, CP), jnp.float32),
)


def _pool_body(table_hbm, textg_hbm, out_hbm, idx_v, rows_v, out_v, *sems):
    wid = lax.axis_index("s") * NC + lax.axis_index("c")
    g0 = wid * SPW
    pltpu.sync_copy(textg_hbm.at[pl.ds(g0, SPW)], idx_v)

    def start(i, s):
        pltpu.make_async_copy(
            table_hbm.at[idx_v.at[i]], rows_v.at[s], sems[s]).start()

    def wait(s):
        pltpu.make_async_copy(
            table_hbm.at[idx_v.at[0]], rows_v.at[s], sems[s]).wait()

    for s in range(NBUF):
        start(s, s)

    def group(gidx, carry):
        for s in range(NBUF):
            i = gidx * NBUF + s
            wait(s)
            for e in range(G):
                def body(l, accs, _e=e):
                    r = _e * H + l
                    return tuple(accs[c] + rows_v[s, r, pl.ds(c * LANES, LANES)]
                                 for c in range(DCH))
                accs = lax.fori_loop(
                    0, H, body,
                    tuple(jnp.zeros((LANES,), jnp.float32) for _ in range(DCH)),
                    unroll=5)
                row_out = i * G + e
                for c in range(DCH):
                    out_v[row_out, pl.ds(c * LANES, LANES)] = accs[c]

            nxt = i + NBUF

            @pl.when(nxt < SPW)
            def _():
                start(nxt, s)
        return carry

    lax.fori_loop(0, NGRP, group, 0)
    pltpu.sync_copy(out_v, out_hbm.at[pl.ds(wid * BPW, BPW)])


_pool = pl.kernel(
    _pool_body,
    out_type=jax.ShapeDtypeStruct((B, D), jnp.float32),
    mesh=plsc.VectorSubcoreMesh(core_axis_name="c", subcore_axis_name="s"),
    scratch_types=[
        pltpu.VMEM((SPW, RPS), jnp.int32),
        pltpu.VMEM((NBUF, RPS, D), jnp.float32),
        pltpu.VMEM((BPW, D), jnp.float32),
    ] + [pltpu.SemaphoreType.DMA] * NBUF,
)


def _fc_body(x_ref, w_ref, b_ref, p_ref, o_ref):
    o_ref[...] = (jnp.dot(x_ref[...], w_ref[...],
                          preferred_element_type=jnp.float32) + b_ref[...]
                  + p_ref[...] * jnp.float32(1e-30))


def kernel(text, emb_table, fc_w, fc_b):
    wp = jnp.zeros((D, CP), jnp.float32).at[:, :C].set(
        fc_w.T * jnp.float32(1.0 / H))
    proj = _proj(emb_table, wp)
    textg = text.astype(jnp.int32).reshape(B // G, RPS)
    pooled = _pool(emb_table, textg)
    wt = fc_w.T * jnp.float32(1.0 / H)
    out = pl.pallas_call(
        _fc_body,
        out_shape=jax.ShapeDtypeStruct((B, C), jnp.float32),
    )(pooled, wt, fc_b.reshape(1, C), proj[:B, :C])
    return out


# speedup vs baseline: 1.4780x; 1.4780x over previous
import jax
import jax.numpy as jnp
from jax import lax
from jax.experimental import pallas as pl
from jax.experimental.pallas import tpu as pltpu
from jax.experimental.pallas import tpu_sc as plsc

B = 4096
H = 50
D = 128
C = 10
LANES = 16
DCH = D // LANES

NC = 2
NS = 16
NW = NC * NS

G = 2
RPS = G * H
SPW = B // G // NW
BPW = B // NW
NBUF = 4
NGRP = SPW // NBUF


def _pool_body(table_hbm, textg_hbm, out_hbm, idx_v, rows_v, out_v, *sems):
    wid = lax.axis_index("s") * NC + lax.axis_index("c")
    g0 = wid * SPW
    pltpu.sync_copy(textg_hbm.at[pl.ds(g0, SPW)], idx_v)

    def start(i, s):
        pltpu.make_async_copy(
            table_hbm.at[idx_v.at[i]], rows_v.at[s], sems[s]).start()

    def wait(s):
        pltpu.make_async_copy(
            table_hbm.at[idx_v.at[0]], rows_v.at[s], sems[s]).wait()

    for s in range(NBUF):
        start(s, s)

    def group(gidx, carry):
        for s in range(NBUF):
            i = gidx * NBUF + s
            wait(s)
            for e in range(G):
                def body(l, accs, _e=e):
                    r = _e * H + l
                    return tuple(accs[c] + rows_v[s, r, pl.ds(c * LANES, LANES)]
                                 for c in range(DCH))
                accs = lax.fori_loop(
                    0, H, body,
                    tuple(jnp.zeros((LANES,), jnp.float32) for _ in range(DCH)),
                    unroll=5)
                row_out = i * G + e
                for c in range(DCH):
                    out_v[row_out, pl.ds(c * LANES, LANES)] = accs[c]

            nxt = i + NBUF

            @pl.when(nxt < SPW)
            def _():
                start(nxt, s)
        return carry

    lax.fori_loop(0, NGRP, group, 0)
    pltpu.sync_copy(out_v, out_hbm.at[pl.ds(wid * BPW, BPW)])


_pool = pl.kernel(
    _pool_body,
    out_type=jax.ShapeDtypeStruct((B, D), jnp.float32),
    mesh=plsc.VectorSubcoreMesh(core_axis_name="c", subcore_axis_name="s"),
    scratch_types=[
        pltpu.VMEM((SPW, RPS), jnp.int32),
        pltpu.VMEM((NBUF, RPS, D), jnp.float32),
        pltpu.VMEM((BPW, D), jnp.float32),
    ] + [pltpu.SemaphoreType.DMA] * NBUF,
    compiler_params=pltpu.CompilerParams(
        disable_bounds_checks=True,
        disable_semaphore_checks=True,
        skip_device_barrier=True,
    ),
)


def _fc_body(x_ref, w_ref, b_ref, o_ref):
    o_ref[...] = jnp.dot(x_ref[...], w_ref[...],
                         preferred_element_type=jnp.float32) + b_ref[...]


def kernel(text, emb_table, fc_w, fc_b):
    textg = text.astype(jnp.int32).reshape(B // G, RPS)
    pooled = _pool(emb_table, textg)
    wt = fc_w.T * jnp.float32(1.0 / H)
    out = pl.pallas_call(
        _fc_body,
        out_shape=jax.ShapeDtypeStruct((B, C), jnp.float32),
    )(pooled, wt, fc_b.reshape(1, C))
    return out


# speedup vs baseline: 1.5024x; 1.0165x over previous
import jax
import jax.numpy as jnp
from jax import lax
from jax.experimental import pallas as pl
from jax.experimental.pallas import tpu as pltpu
from jax.experimental.pallas import tpu_sc as plsc

B = 4096
H = 50
D = 128
C = 10
LANES = 16
DCH = D // LANES

NC = 2
NS = 16
NW = NC * NS

G = 2
RPS = G * H
SPW = B // G // NW
BPW = B // NW
NBUF = 6
NGRP = SPW // NBUF


def _pool_body(table_hbm, textg_hbm, out_hbm, idx_v, rows_v, out_v, *sems):
    wid = lax.axis_index("s") * NC + lax.axis_index("c")
    g0 = wid * SPW
    pltpu.sync_copy(textg_hbm.at[pl.ds(g0, SPW)], idx_v)

    def start(i, s):
        pltpu.make_async_copy(
            table_hbm.at[idx_v.at[i]], rows_v.at[s], sems[s]).start()

    def wait(s):
        pltpu.make_async_copy(
            table_hbm.at[idx_v.at[0]], rows_v.at[s], sems[s]).wait()

    for s in range(NBUF):
        start(s, s)

    def group(gidx, carry):
        for s in range(NBUF):
            i = gidx * NBUF + s
            wait(s)
            for e in range(G):
                def body(l, accs, _e=e):
                    r = _e * H + l
                    return tuple(accs[c] + rows_v[s, r, pl.ds(c * LANES, LANES)]
                                 for c in range(DCH))
                accs = lax.fori_loop(
                    0, H, body,
                    tuple(jnp.zeros((LANES,), jnp.float32) for _ in range(DCH)),
                    unroll=5)
                row_out = i * G + e
                for c in range(DCH):
                    out_v[row_out, pl.ds(c * LANES, LANES)] = accs[c]

            nxt = i + NBUF

            @pl.when(nxt < SPW)
            def _():
                start(nxt, s)
        return carry

    lax.fori_loop(0, NGRP, group, 0)
    pltpu.sync_copy(out_v, out_hbm.at[pl.ds(wid * BPW, BPW)])


_pool = pl.kernel(
    _pool_body,
    out_type=jax.ShapeDtypeStruct((B, D), jnp.float32),
    mesh=plsc.VectorSubcoreMesh(core_axis_name="c", subcore_axis_name="s"),
    scratch_types=[
        pltpu.VMEM((SPW, RPS), jnp.int32),
        pltpu.VMEM((NBUF, RPS, D), jnp.float32),
        pltpu.VMEM((BPW, D), jnp.float32),
    ] + [pltpu.SemaphoreType.DMA] * NBUF,
    compiler_params=pltpu.CompilerParams(
        disable_bounds_checks=True,
        disable_semaphore_checks=True,
        skip_device_barrier=True,
    ),
)


def _fc_body(x_ref, w_ref, b_ref, o_ref):
    o_ref[...] = jnp.dot(x_ref[...], w_ref[...],
                         preferred_element_type=jnp.float32) + b_ref[...]


def kernel(text, emb_table, fc_w, fc_b):
    textg = text.astype(jnp.int32).reshape(B // G, RPS)
    pooled = _pool(emb_table, textg)
    wt = fc_w.T * jnp.float32(1.0 / H)
    out = pl.pallas_call(
        _fc_body,
        out_shape=jax.ShapeDtypeStruct((B, C), jnp.float32),
    )(pooled, wt, fc_b.reshape(1, C))
    return out
